# baseline (speedup 1.0000x reference)
import jax
import jax.numpy as jnp
from jax.experimental import pallas as pl


def _gat_block_kernel(x_ref, w_ref, att_ref, bias_ref, out_ref):
    a, s_dim, _, two, d = x_ref.shape
    g = s_dim * two
    xb = x_ref[...].reshape(a, s_dim, two, d)
    xm = jnp.transpose(xb, (2, 1, 0, 3)).reshape(g, a, d)

    w = w_ref[...]
    h = jnp.dot(
        xm.reshape(g * a, d), w, preferred_element_type=jnp.float32
    ).reshape(g, a, d)

    att = att_ref[...]
    a_src = jnp.sum(h * att[0][None, None, :], axis=-1)
    a_dst = jnp.sum(h * att[1][None, None, :], axis=-1)

    e = a_src[:, :, None] + a_dst[:, None, :]
    e = jnp.maximum(e, 0.2 * e)
    ex = jnp.exp(e)
    denom = jnp.sum(ex, axis=1, keepdims=True)
    alpha = ex / denom

    agg = jax.lax.dot_general(
        alpha, h, (((1,), (1,)), ((0,), (0,))),
        preferred_element_type=jnp.float32,
    )

    out = agg + bias_ref[0][None, None, :] + xm
    out = jnp.transpose(out, (1, 0, 2))
    out_ref[...] = out.reshape(a, 4, g // 4, d)


def kernel(embeddings, W, att_src, att_dst, bias):
    a, s, p, d = embeddings.shape
    emb5 = embeddings.reshape(a, s, p // 8, 8, d)
    grid = (p // 8,)
    out = pl.pallas_call(
        _gat_block_kernel,
        grid=grid,
        in_specs=[
            pl.BlockSpec((a, s, 1, 8, d), lambda u: (0, 0, u, 0, 0)),
            pl.BlockSpec((d, d), lambda u: (0, 0)),
            pl.BlockSpec((2, d), lambda u: (0, 0)),
            pl.BlockSpec((1, d), lambda u: (0, 0)),
        ],
        out_specs=pl.BlockSpec((a, 4, 2 * s, d), lambda u: (0, u, 0, 0)),
        out_shape=jax.ShapeDtypeStruct((a, p // 2, 2 * s, d), jnp.float32),
    )(
        emb5,
        W,
        jnp.stack([att_src, att_dst], axis=0),
        bias.reshape(1, d),
    )
    return out


# speedup vs baseline: 1.0008x; 1.0008x over previous
import jax
import jax.numpy as jnp
from jax.experimental import pallas as pl
from jax.experimental.pallas import tpu as pltpu


def _gat_block_kernel(x_ref, w_ref, att_ref, bias_ref, out_ref):
    a, s_dim, _, two, d = x_ref.shape
    g = s_dim * two
    xb = x_ref[...].reshape(a, s_dim, two, d)
    xm = jnp.transpose(xb, (2, 1, 0, 3)).reshape(g, a, d)

    w = w_ref[...]
    h = jnp.dot(
        xm.reshape(g * a, d), w, preferred_element_type=jnp.float32
    ).reshape(g, a, d)

    att = att_ref[...]
    a_src = jnp.sum(h * att[0][None, None, :], axis=-1)
    a_dst = jnp.sum(h * att[1][None, None, :], axis=-1)

    e = a_src[:, :, None] + a_dst[:, None, :]
    e = jnp.maximum(e, 0.2 * e)
    ex = jnp.exp(e)
    denom = jnp.sum(ex, axis=1, keepdims=True)
    alpha = ex / denom

    agg = jax.lax.dot_general(
        alpha, h, (((1,), (1,)), ((0,), (0,))),
        preferred_element_type=jnp.float32,
    )

    out = agg + bias_ref[0][None, None, :] + xm
    out = jnp.transpose(out, (1, 0, 2))
    out_ref[...] = out.reshape(a, 4, g // 4, d)


def kernel(embeddings, W, att_src, att_dst, bias):
    a, s, p, d = embeddings.shape
    emb5 = embeddings.reshape(a, s, p // 8, 8, d)
    grid = (p // 8,)
    out = pl.pallas_call(
        _gat_block_kernel,
        grid=grid,
        in_specs=[
            pl.BlockSpec((a, s, 1, 8, d), lambda u: (0, 0, u, 0, 0)),
            pl.BlockSpec((d, d), lambda u: (0, 0)),
            pl.BlockSpec((2, d), lambda u: (0, 0)),
            pl.BlockSpec((1, d), lambda u: (0, 0)),
        ],
        out_specs=pl.BlockSpec((a, 4, 2 * s, d), lambda u: (0, u, 0, 0)),
        out_shape=jax.ShapeDtypeStruct((a, p // 2, 2 * s, d), jnp.float32),
        compiler_params=pltpu.CompilerParams(
            dimension_semantics=("parallel",),
        ),
    )(
        emb5,
        W,
        jnp.stack([att_src, att_dst], axis=0),
        bias.reshape(1, d),
    )
    return out


# speedup vs baseline: 1.0492x; 1.0483x over previous
import jax
import jax.numpy as jnp
from jax.experimental import pallas as pl


def _gat_block_kernel(x_ref, w_ref, asrc_ref, adst_ref, bias_ref, out_ref):
    a, s_dim, _, two, d = x_ref.shape
    g = s_dim * two
    xb = x_ref[...].reshape(a, s_dim, two, d)
    xm = jnp.transpose(xb, (2, 1, 0, 3)).reshape(g, a, d)

    w = w_ref[...]
    h = jnp.dot(
        xm.reshape(g * a, d), w, preferred_element_type=jnp.float32
    ).reshape(g, a, d)

    a_src = jnp.sum(h * asrc_ref[0][None, None, :], axis=-1)
    a_dst = jnp.sum(h * adst_ref[0][None, None, :], axis=-1)

    e = a_src[:, :, None] + a_dst[:, None, :]
    e = jnp.maximum(e, 0.2 * e)
    m = jnp.max(e, axis=1, keepdims=True)
    ex = jnp.exp(e - m)
    denom = jnp.sum(ex, axis=1, keepdims=True)
    alpha = ex / denom

    agg = jax.lax.dot_general(
        alpha, h, (((1,), (1,)), ((0,), (0,))),
        preferred_element_type=jnp.float32,
    )

    out = agg + bias_ref[0][None, None, :] + xm
    out = jnp.transpose(out, (1, 0, 2))
    out_ref[...] = out.reshape(a, 4, g // 4, d)


def kernel(embeddings, W, att_src, att_dst, bias):
    a, s, p, d = embeddings.shape
    emb5 = embeddings.reshape(a, s, p // 8, 8, d)
    grid = (p // 8,)
    out = pl.pallas_call(
        _gat_block_kernel,
        grid=grid,
        in_specs=[
            pl.BlockSpec((a, s, 1, 8, d), lambda u: (0, 0, u, 0, 0)),
            pl.BlockSpec((d, d), lambda u: (0, 0)),
            pl.BlockSpec((1, d), lambda u: (0, 0)),
            pl.BlockSpec((1, d), lambda u: (0, 0)),
            pl.BlockSpec((1, d), lambda u: (0, 0)),
        ],
        out_specs=pl.BlockSpec((a, 4, 2 * s, d), lambda u: (0, u, 0, 0)),
        out_shape=jax.ShapeDtypeStruct((a, p // 2, 2 * s, d), jnp.float32),
    )(
        emb5,
        W,
        att_src.reshape(1, d),
        att_dst.reshape(1, d),
        bias.reshape(1, d),
    )
    return out
